# baseline (speedup 1.0000x reference)
import jax
import jax.numpy as jnp
from jax import lax
from jax.experimental import pallas as pl
from jax.experimental.pallas import tpu as pltpu

H = 128
B = 16
N = 16384
NB = 2048
NXC = N // NB
VT = 6272
RING = 10

_V = 100000
_TILES = [(i * VT, min(VT, _V - i * VT)) for i in range((_V + VT - 1) // VT)]
_NT = len(_TILES)


def _tbl_copy(tbl_hbm, ring, tsem, j):
    off, w = _TILES[j]
    return pltpu.make_async_copy(tbl_hbm.at[pl.ds(off, w), :],
                                 ring.at[j % RING, pl.ds(0, w), :],
                                 tsem.at[j % RING])


def _x_copy(x_hbm, x_vmem, xsem, k):
    return pltpu.make_async_copy(x_hbm.at[pl.ds(k * NB, NB), :],
                                 x_vmem.at[pl.ds(k * NB, NB), :],
                                 xsem.at[k])


def _prep(x_hbm, batch_ref, nc_ref, w1_ref, b1_ref, w2_ref, b2_ref,
          qwt_ref, qb_ref, w3_ref, b3_ref, tbl_hbm,
          x_vmem, ring, s_h_ref, tsem, xsem):
    for j in range(min(RING, _NT)):
        _tbl_copy(tbl_hbm, ring, tsem, j).start()
    for k in range(NXC):
        _x_copy(x_hbm, x_vmem, xsem, k).start()

    batch = batch_ref[:, :]
    seg = lax.broadcasted_iota(jnp.int32, (B, N), 0)
    onehot_t = (batch == seg)
    pos = lax.broadcasted_iota(jnp.int32, (B, N), 1)
    masked = jnp.where(onehot_t, pos, -1)
    last = jnp.max(masked, axis=1, keepdims=True)
    last = jnp.clip(last, 0, N - 1)
    lastoh_t = (pos == last).astype(jnp.float32)
    onehot_f = onehot_t.astype(jnp.float32)

    v_n = jnp.zeros((B, H), dtype=jnp.float32)
    for k in range(NXC):
        _x_copy(x_hbm, x_vmem, xsem, k).wait()
        xk = x_vmem[k * NB:(k + 1) * NB, :]
        v_n = v_n + jnp.dot(lastoh_t[:, k * NB:(k + 1) * NB], xk,
                            preferred_element_type=jnp.float32)

    c = (jnp.dot(v_n, w1_ref[:, :], preferred_element_type=jnp.float32)
         + b1_ref[:, :] + b2_ref[:, :])
    qb = qb_ref[0, 0]
    w2 = w2_ref[:, :]
    qwt = qwt_ref[:, :]

    s_g = jnp.zeros((B, H), dtype=jnp.float32)
    for k in range(NXC):
        xk = x_vmem[k * NB:(k + 1) * NB, :]
        oh_k = onehot_f[:, k * NB:(k + 1) * NB]
        cb_k = lax.dot_general(oh_k, c, (((0,), (0,)), ((), ())),
                               preferred_element_type=jnp.float32)
        pre = jnp.dot(xk, w2, preferred_element_type=jnp.float32) + cb_k
        sg = jax.nn.sigmoid(pre)
        alpha = lax.dot_general(qwt, sg, (((1,), (1,)), ((), ())),
                                preferred_element_type=jnp.float32) + qb
        wk = nc_ref[:, k * NB:(k + 1) * NB] * alpha
        a_k = oh_k * wk
        s_g = s_g + jnp.dot(a_k, xk, preferred_element_type=jnp.float32)

    s_h_ref[:, :] = (
        jnp.dot(v_n, w3_ref[0:H, :], preferred_element_type=jnp.float32)
        + jnp.dot(s_g, w3_ref[H:2 * H, :], preferred_element_type=jnp.float32)
        + b3_ref[:, :])


def _body(x_hbm, batch_ref, nc_ref, w1_ref, b1_ref, w2_ref, b2_ref,
          qwt_ref, qb_ref, w3_ref, b3_ref, tbl_hbm, out_ref,
          x_vmem, ring, s_h_ref, tsem, xsem):
    @pl.when(pl.program_id(0) == 0)
    def _():
        _prep(x_hbm, batch_ref, nc_ref, w1_ref, b1_ref, w2_ref, b2_ref,
              qwt_ref, qb_ref, w3_ref, b3_ref, tbl_hbm,
              x_vmem, ring, s_h_ref, tsem, xsem)

    for j in range(_NT):
        @pl.when(pl.program_id(0) == j)
        def _(j=j):
            off, w = _TILES[j]
            slot = j % RING
            _tbl_copy(tbl_hbm, ring, tsem, j).wait()
            t = ring[slot, 0:w, :]
            zj = lax.dot_general(s_h_ref[:, :], t, (((1,), (1,)), ((), ())),
                                 preferred_element_type=jnp.float32)
            out_ref[:, 0:w] = zj
            if j + RING < _NT:
                _tbl_copy(tbl_hbm, ring, tsem, j + RING).start()


@jax.jit
def kernel(node_embedding, item_embedding_table, batch, num_count,
           W1, b1, W2, b2, qw, qb, W3, b3):
    n, h = node_embedding.shape
    v = item_embedding_table.shape[0]
    batch_row = batch.astype(jnp.int32).reshape(1, n)
    nc_row = num_count.reshape(1, n)

    const = lambda i: (0, 0)
    vmem = lambda: pl.BlockSpec(memory_space=pltpu.MemorySpace.VMEM)
    hbm = lambda: pl.BlockSpec(memory_space=pltpu.MemorySpace.HBM)
    z = pl.pallas_call(
        _body,
        grid=(_NT,),
        in_specs=[hbm(),
                  pl.BlockSpec((1, n), const),
                  pl.BlockSpec((1, n), const),
                  pl.BlockSpec((h, h), const),
                  pl.BlockSpec((1, h), const),
                  pl.BlockSpec((h, h), const),
                  pl.BlockSpec((1, h), const),
                  pl.BlockSpec((1, h), const),
                  pl.BlockSpec((1, 1), const),
                  pl.BlockSpec((2 * h, h), const),
                  pl.BlockSpec((1, h), const),
                  hbm()],
        out_specs=pl.BlockSpec((B, VT), lambda i: (0, i)),
        out_shape=jax.ShapeDtypeStruct((B, v), jnp.float32),
        scratch_shapes=[
            pltpu.VMEM((N, H), jnp.float32),
            pltpu.VMEM((RING, VT, H), jnp.float32),
            pltpu.VMEM((B, H), jnp.float32),
            pltpu.SemaphoreType.DMA((RING,)),
            pltpu.SemaphoreType.DMA((NXC,)),
        ],
    )(node_embedding, batch_row, nc_row,
      W1, b1.reshape(1, h), W2, b2.reshape(1, h),
      qw.reshape(1, h), qb.reshape(1, 1), W3, b3.reshape(1, h),
      item_embedding_table)
    return z


# speedup vs baseline: 1.0111x; 1.0111x over previous
import jax
import jax.numpy as jnp
from jax import lax
from jax.experimental import pallas as pl
from jax.experimental.pallas import tpu as pltpu

H = 128
B = 16
N = 16384
NB = 2048
NXC = N // NB
VT = 6272
RING = 10

_V = 100000
_TILES = [(i * VT, min(VT, _V - i * VT)) for i in range((_V + VT - 1) // VT)]
_NT = len(_TILES)


def _tbl_copy(tbl_hbm, ring, tsem, j):
    off, w = _TILES[j]
    return pltpu.make_async_copy(tbl_hbm.at[pl.ds(off, w), :],
                                 ring.at[j % RING, pl.ds(0, w), :],
                                 tsem.at[j % RING])


def _x_copy(x_hbm, x_vmem, xsem, k):
    return pltpu.make_async_copy(x_hbm.at[pl.ds(k * NB, NB), :],
                                 x_vmem.at[pl.ds(k * NB, NB), :],
                                 xsem.at[k])


def _prep(x_hbm, batch_ref, nc_ref, w1_ref, b1_ref, w2_ref, b2_ref,
          qwt_ref, qb_ref, w3_ref, b3_ref, tbl_hbm,
          x_vmem, ring, s_h_ref, tsem, xsem):
    for k in range(NXC):
        _x_copy(x_hbm, x_vmem, xsem, k).start()
    for j in range(min(RING, _NT)):
        _tbl_copy(tbl_hbm, ring, tsem, j).start()

    batch = batch_ref[:, :]
    seg = lax.broadcasted_iota(jnp.int32, (B, N), 0)
    onehot_t = (batch == seg)
    pos = lax.broadcasted_iota(jnp.int32, (B, N), 1)
    masked = jnp.where(onehot_t, pos, -1)
    last = jnp.max(masked, axis=1, keepdims=True)
    last = jnp.clip(last, 0, N - 1)
    lastoh_t = (pos == last).astype(jnp.float32)
    onehot_f = onehot_t.astype(jnp.float32)

    v_n = jnp.zeros((B, H), dtype=jnp.float32)
    for k in range(NXC):
        _x_copy(x_hbm, x_vmem, xsem, k).wait()
        xk = x_vmem[k * NB:(k + 1) * NB, :]
        v_n = v_n + jnp.dot(lastoh_t[:, k * NB:(k + 1) * NB], xk,
                            preferred_element_type=jnp.float32)

    c = (jnp.dot(v_n, w1_ref[:, :], preferred_element_type=jnp.float32)
         + b1_ref[:, :] + b2_ref[:, :])
    qb = qb_ref[0, 0]
    w2 = w2_ref[:, :]
    qwt = qwt_ref[:, :]

    s_g = jnp.zeros((B, H), dtype=jnp.float32)
    for k in range(NXC):
        xk = x_vmem[k * NB:(k + 1) * NB, :]
        oh_k = onehot_f[:, k * NB:(k + 1) * NB]
        cb_k = lax.dot_general(oh_k, c, (((0,), (0,)), ((), ())),
                               preferred_element_type=jnp.float32)
        pre = jnp.dot(xk, w2, preferred_element_type=jnp.float32) + cb_k
        sg = jax.nn.sigmoid(pre)
        alpha = lax.dot_general(qwt, sg, (((1,), (1,)), ((), ())),
                                preferred_element_type=jnp.float32) + qb
        wk = nc_ref[:, k * NB:(k + 1) * NB] * alpha
        a_k = oh_k * wk
        s_g = s_g + jnp.dot(a_k, xk, preferred_element_type=jnp.float32)

    s_h_ref[:, :] = (
        jnp.dot(v_n, w3_ref[0:H, :], preferred_element_type=jnp.float32)
        + jnp.dot(s_g, w3_ref[H:2 * H, :], preferred_element_type=jnp.float32)
        + b3_ref[:, :])


def _body(x_hbm, batch_ref, nc_ref, w1_ref, b1_ref, w2_ref, b2_ref,
          qwt_ref, qb_ref, w3_ref, b3_ref, tbl_hbm, out_ref,
          x_vmem, ring, s_h_ref, tsem, xsem):
    @pl.when(pl.program_id(0) == 0)
    def _():
        _prep(x_hbm, batch_ref, nc_ref, w1_ref, b1_ref, w2_ref, b2_ref,
              qwt_ref, qb_ref, w3_ref, b3_ref, tbl_hbm,
              x_vmem, ring, s_h_ref, tsem, xsem)

    for j in range(_NT):
        @pl.when(pl.program_id(0) == j)
        def _(j=j):
            off, w = _TILES[j]
            slot = j % RING
            _tbl_copy(tbl_hbm, ring, tsem, j).wait()
            t = ring[slot, 0:w, :]
            zj = lax.dot_general(s_h_ref[:, :], t, (((1,), (1,)), ((), ())),
                                 preferred_element_type=jnp.float32)
            out_ref[:, 0:w] = zj
            if j + RING < _NT:
                _tbl_copy(tbl_hbm, ring, tsem, j + RING).start()


@jax.jit
def kernel(node_embedding, item_embedding_table, batch, num_count,
           W1, b1, W2, b2, qw, qb, W3, b3):
    n, h = node_embedding.shape
    v = item_embedding_table.shape[0]
    batch_row = batch.astype(jnp.int32).reshape(1, n)
    nc_row = num_count.reshape(1, n)

    const = lambda i: (0, 0)
    vmem = lambda: pl.BlockSpec(memory_space=pltpu.MemorySpace.VMEM)
    hbm = lambda: pl.BlockSpec(memory_space=pltpu.MemorySpace.HBM)
    z = pl.pallas_call(
        _body,
        grid=(_NT,),
        in_specs=[hbm(),
                  pl.BlockSpec((1, n), const),
                  pl.BlockSpec((1, n), const),
                  pl.BlockSpec((h, h), const),
                  pl.BlockSpec((1, h), const),
                  pl.BlockSpec((h, h), const),
                  pl.BlockSpec((1, h), const),
                  pl.BlockSpec((1, h), const),
                  pl.BlockSpec((1, 1), const),
                  pl.BlockSpec((2 * h, h), const),
                  pl.BlockSpec((1, h), const),
                  hbm()],
        out_specs=pl.BlockSpec((B, VT), lambda i: (0, i)),
        out_shape=jax.ShapeDtypeStruct((B, v), jnp.float32),
        scratch_shapes=[
            pltpu.VMEM((N, H), jnp.float32),
            pltpu.VMEM((RING, VT, H), jnp.float32),
            pltpu.VMEM((B, H), jnp.float32),
            pltpu.SemaphoreType.DMA((RING,)),
            pltpu.SemaphoreType.DMA((NXC,)),
        ],
    )(node_embedding, batch_row, nc_row,
      W1, b1.reshape(1, h), W2, b2.reshape(1, h),
      qw.reshape(1, h), qb.reshape(1, 1), W3, b3.reshape(1, h),
      item_embedding_table)
    return z
